# baseline (speedup 1.0000x reference)
import jax
import jax.numpy as jnp
from jax.experimental import pallas as pl
from jax.experimental.pallas import tpu as pltpu

HIDDEN = 128
NUM_CLASSES = 10
C_PAD = 128
B_TILE = 256
N_SPLIT = 2


def _lstm_kernel(x_ref, w1_ref, b1_ref, w2_ref, b2_ref,
                 wfc_ref, bfc_ref, out_ref):
    T, Bt, _ = x_ref.shape
    H = w2_ref.shape[0] // 2

    w1 = w1_ref[...]
    w2 = w2_ref[...]
    b1 = b1_ref[...]
    b2 = b2_ref[...]

    def activate(gates):
        i = 0.5 * jnp.tanh(gates[:, 0 * H:1 * H]) + 0.5
        f = 0.5 * jnp.tanh(gates[:, 1 * H:2 * H]) + 0.5
        g = jnp.tanh(gates[:, 2 * H:3 * H])
        o = 0.5 * jnp.tanh(gates[:, 3 * H:4 * H]) + 0.5
        return i, f, g, o

    ns = N_SPLIT
    Bs = Bt // ns
    h1 = [jnp.zeros((Bs, H), jnp.float32) for _ in range(ns)]
    c1 = [jnp.zeros((Bs, H), jnp.float32) for _ in range(ns)]
    h2 = [jnp.zeros((Bs, H), jnp.float32) for _ in range(ns)]
    c2 = [jnp.zeros((Bs, H), jnp.float32) for _ in range(ns)]

    for t in range(T):
        for s in range(ns):
            xt = x_ref[t, s * Bs:(s + 1) * Bs, :]
            lhs1 = jnp.concatenate([h1[s].astype(x_ref.dtype), xt], axis=1)
            g1 = (jnp.dot(lhs1, w1, preferred_element_type=jnp.float32) + b1)
            i1, f1, gg1, o1 = activate(g1)
            c1[s] = f1 * c1[s] + i1 * gg1
            h1[s] = o1 * jnp.tanh(c1[s])

            g2 = jnp.dot(jnp.concatenate([h1[s], h2[s]], axis=1), w2,
                         preferred_element_type=jnp.float32) + b2
            i2, f2, gg2, o2 = activate(g2)
            c2[s] = f2 * c2[s] + i2 * gg2
            h2[s] = o2 * jnp.tanh(c2[s])

    h2_all = jnp.concatenate(h2, axis=0)
    out_ref[...] = (jnp.dot(h2_all, wfc_ref[...],
                            preferred_element_type=jnp.float32)
                    + bfc_ref[...])


def kernel(x, w_ih_l0, w_hh_l0, b_l0, w_ih_l1, w_hh_l1, b_l1, w_fc, b_fc):
    B, T, I = x.shape
    H = w_hh_l0.shape[1]
    C = w_fc.shape[0]

    gate_scale = jnp.concatenate([
        jnp.full((2 * H,), 0.5, jnp.float32),
        jnp.ones((H,), jnp.float32),
        jnp.full((H,), 0.5, jnp.float32),
    ])[None, :]
    w1 = (jnp.concatenate([w_hh_l0.T, w_ih_l0.T], axis=0)
          * gate_scale).astype(jnp.bfloat16)
    b1 = (b_l0[None, :] * gate_scale).astype(jnp.float32)
    w2 = (jnp.concatenate([w_ih_l1.T, w_hh_l1.T], axis=0)
          * gate_scale).astype(jnp.float32)
    b2 = (b_l1[None, :] * gate_scale).astype(jnp.float32)
    wfc = jnp.zeros((H, C_PAD), jnp.float32).at[:, :C].set(w_fc.T)
    bfc = jnp.zeros((1, C_PAD), jnp.float32).at[:, :C].set(b_fc[None, :])

    x_tbi = jnp.transpose(x, (1, 0, 2)).astype(jnp.bfloat16)

    n_btiles = B // B_TILE
    const = lambda b: (0, 0)

    out = pl.pallas_call(
        _lstm_kernel,
        out_shape=jax.ShapeDtypeStruct((B, C_PAD), jnp.float32),
        grid=(n_btiles,),
        in_specs=[
            pl.BlockSpec((T, B_TILE, I), lambda b: (0, b, 0)),
            pl.BlockSpec((H + I, 4 * H), const),
            pl.BlockSpec((1, 4 * H), const),
            pl.BlockSpec((2 * H, 4 * H), const),
            pl.BlockSpec((1, 4 * H), const),
            pl.BlockSpec((H, C_PAD), const),
            pl.BlockSpec((1, C_PAD), const),
        ],
        out_specs=pl.BlockSpec((B_TILE, C_PAD), lambda b: (b, 0)),
        compiler_params=pltpu.CompilerParams(
            dimension_semantics=("parallel",)),
    )(x_tbi, w1, b1, w2, b2, wfc, bfc)

    return out[:, :C]


# speedup vs baseline: 1.8792x; 1.8792x over previous
import jax
import jax.numpy as jnp
from jax.experimental import pallas as pl
from jax.experimental.pallas import tpu as pltpu

HIDDEN = 128
NUM_CLASSES = 10
C_PAD = 128
B_TILE = 512
N_SPLIT = 1


def _lstm_kernel(x_ref, w1_ref, b1_ref, w2_ref, b2_ref,
                 wfc_ref, bfc_ref, out_ref):
    T, Bt, _ = x_ref.shape
    H = w2_ref.shape[0] // 2

    w1 = w1_ref[...]
    w2 = w2_ref[...]
    b1 = b1_ref[...]
    b2 = b2_ref[...]

    def activate(gates):
        i = 0.5 * jnp.tanh(gates[:, 0 * H:1 * H]) + 0.5
        f = 0.5 * jnp.tanh(gates[:, 1 * H:2 * H]) + 0.5
        g = jnp.tanh(gates[:, 2 * H:3 * H])
        o = 0.5 * jnp.tanh(gates[:, 3 * H:4 * H]) + 0.5
        return i, f, g, o

    ns = N_SPLIT
    Bs = Bt // ns
    h1 = [jnp.zeros((Bs, H), jnp.float32) for _ in range(ns)]
    c1 = [jnp.zeros((Bs, H), jnp.float32) for _ in range(ns)]
    h2 = [jnp.zeros((Bs, H), jnp.float32) for _ in range(ns)]
    c2 = [jnp.zeros((Bs, H), jnp.float32) for _ in range(ns)]

    for t in range(T):
        for s in range(ns):
            xt = x_ref[t, s * Bs:(s + 1) * Bs, :]
            lhs1 = jnp.concatenate([h1[s].astype(x_ref.dtype), xt], axis=1)
            g1 = (jnp.dot(lhs1, w1, preferred_element_type=jnp.float32) + b1)
            i1, f1, gg1, o1 = activate(g1)
            c1[s] = f1 * c1[s] + i1 * gg1
            h1[s] = o1 * jnp.tanh(c1[s])

            g2 = jnp.dot(jnp.concatenate([h1[s], h2[s]], axis=1), w2,
                         preferred_element_type=jnp.float32) + b2
            i2, f2, gg2, o2 = activate(g2)
            c2[s] = f2 * c2[s] + i2 * gg2
            h2[s] = o2 * jnp.tanh(c2[s])

    h2_all = jnp.concatenate(h2, axis=0)
    out_ref[...] = (jnp.dot(h2_all, wfc_ref[...],
                            preferred_element_type=jnp.float32)
                    + bfc_ref[...])


def kernel(x, w_ih_l0, w_hh_l0, b_l0, w_ih_l1, w_hh_l1, b_l1, w_fc, b_fc):
    B, T, I = x.shape
    H = w_hh_l0.shape[1]
    C = w_fc.shape[0]

    gate_scale = jnp.concatenate([
        jnp.full((2 * H,), 0.5, jnp.float32),
        jnp.ones((H,), jnp.float32),
        jnp.full((H,), 0.5, jnp.float32),
    ])[None, :]
    w1 = (jnp.concatenate([w_hh_l0.T, w_ih_l0.T], axis=0)
          * gate_scale).astype(jnp.bfloat16)
    b1 = (b_l0[None, :] * gate_scale).astype(jnp.float32)
    w2 = (jnp.concatenate([w_ih_l1.T, w_hh_l1.T], axis=0)
          * gate_scale).astype(jnp.float32)
    b2 = (b_l1[None, :] * gate_scale).astype(jnp.float32)
    wfc = jnp.zeros((H, C_PAD), jnp.float32).at[:, :C].set(w_fc.T)
    bfc = jnp.zeros((1, C_PAD), jnp.float32).at[:, :C].set(b_fc[None, :])

    x_tbi = jnp.transpose(x, (1, 0, 2)).astype(jnp.bfloat16)

    n_btiles = B // B_TILE
    const = lambda b: (0, 0)

    out = pl.pallas_call(
        _lstm_kernel,
        out_shape=jax.ShapeDtypeStruct((B, C_PAD), jnp.float32),
        grid=(n_btiles,),
        in_specs=[
            pl.BlockSpec((T, B_TILE, I), lambda b: (0, b, 0)),
            pl.BlockSpec((H + I, 4 * H), const),
            pl.BlockSpec((1, 4 * H), const),
            pl.BlockSpec((2 * H, 4 * H), const),
            pl.BlockSpec((1, 4 * H), const),
            pl.BlockSpec((H, C_PAD), const),
            pl.BlockSpec((1, C_PAD), const),
        ],
        out_specs=pl.BlockSpec((B_TILE, C_PAD), lambda b: (b, 0)),
        compiler_params=pltpu.CompilerParams(
            dimension_semantics=("parallel",)),
    )(x_tbi, w1, b1, w2, b2, wfc, bfc)

    return out[:, :C]
